# baseline (speedup 1.0000x reference)
import jax
import jax.numpy as jnp
from jax import lax
from jax.experimental import pallas as pl
from jax.experimental.pallas import tpu as pltpu
from jax.experimental.pallas import tpu_sc as plsc

NUM_CLASSES = 1000
D = 768
N = 16384
EPS = 1e-5

NC = 2
NS = 16
L = 16
NW = NC * NS
CPW = 32
ACC_ROWS = NW * CPW
DP = D + 128
CHUNK = 64
DL = D // L
LOG2N = 14


def _sc_body(embs_hbm, ids_hbm, gam_hbm, bet_hbm, out_hbm,
             ids_full, in_buf, idx_v, gam_v, bet_v, acc):
    c = lax.axis_index("c")
    s = lax.axis_index("s")
    wid = c * NS + s
    base = wid * CPW

    pltpu.sync_copy(gam_hbm, gam_v)
    pltpu.sync_copy(bet_hbm, bet_v)
    pltpu.sync_copy(ids_hbm, ids_full)

    zero = jnp.zeros((L,), jnp.float32)

    def _zr(r, _):
        def _zc(j, _):
            acc[r, pl.ds(j * L, L)] = zero
            return 0
        lax.fori_loop(0, DP // L, _zc, 0)
        return 0
    lax.fori_loop(0, CPW, _zr, 0)

    def _sload(ref, i):
        return jnp.max(plsc.load_gather(ref, [jnp.full((L,), i, jnp.int32)]))

    def _lower_bound(target):
        def step(_, lohi):
            lo, hi = lohi
            mid = (lo + hi) // 2
            pred = _sload(ids_full, mid) < target
            return jnp.where(pred, mid + 1, lo), jnp.where(pred, hi, mid)
        lo, _ = lax.fori_loop(0, LOG2N, step, (jnp.int32(0), jnp.int32(N)))
        return lo

    row_start = _lower_bound(base)
    row_end = _lower_bound(base + CPW)
    ck_lo = row_start // CHUNK
    ck_hi = (row_end + CHUNK - 1) // CHUNK

    cnt_vec = (lax.iota(jnp.int32, L) == 0).astype(jnp.float32)
    inv_d = jnp.float32(1.0 / D)

    def chunk_body(ck, _):
        g0 = ck * CHUNK
        pltpu.sync_copy(ids_hbm.at[pl.ds(g0, CHUNK)], idx_v)
        pltpu.sync_copy(embs_hbm.at[pl.ds(g0, CHUNK)], in_buf)
        r_hi = jnp.minimum(row_end - g0, CHUNK)
        r_lo = jnp.minimum(jnp.maximum(row_start - g0, 0), r_hi)

        def row_body(r, _):
            lid = _sload(idx_v, r) - base

            def p1(j, carry):
                sv, s2 = carry
                v = in_buf[r, pl.ds(j * L, L)]
                return sv + v, s2 + v * v
            sv, s2 = lax.fori_loop(0, DL, p1, (zero, zero))
            mean = jnp.sum(sv) * inv_d
            var = jnp.sum(s2) * inv_d - mean * mean
            vv = jnp.full((L,), var + EPS, dtype=jnp.float32)
            yi = lax.bitcast_convert_type(vv, jnp.int32)
            yi = 0x5F3759DF - (yi >> 1)
            y = lax.bitcast_convert_type(yi, jnp.float32)
            half = vv * 0.5
            for _i in range(4):
                y = y * (1.5 - half * y * y)
            meanv = jnp.full((L,), mean, dtype=jnp.float32)

            def p2(j, _):
                v = in_buf[r, pl.ds(j * L, L)]
                g = gam_v[pl.ds(j * L, L)]
                b = bet_v[pl.ds(j * L, L)]
                xn = (v - meanv) * y * g + b
                acc[lid, pl.ds(j * L, L)] = acc[lid, pl.ds(j * L, L)] + xn
                return 0
            lax.fori_loop(0, DL, p2, 0)
            acc[lid, pl.ds(D, L)] = acc[lid, pl.ds(D, L)] + cnt_vec
            return 0
        lax.fori_loop(r_lo, r_hi, row_body, 0)
        return 0
    lax.fori_loop(jnp.minimum(ck_lo, ck_hi), ck_hi, chunk_body, 0)

    pltpu.sync_copy(acc, out_hbm.at[pl.ds(base, CPW)])


def _sc_segment_ln(embs, ids, ln_gamma, ln_beta):
    mesh = plsc.VectorSubcoreMesh(core_axis_name="c", subcore_axis_name="s",
                                  num_cores=NC, num_subcores=NS)
    return pl.kernel(
        _sc_body,
        out_type=jax.ShapeDtypeStruct((ACC_ROWS, DP), jnp.float32),
        mesh=mesh,
        compiler_params=pltpu.CompilerParams(needs_layout_passes=False),
        scratch_types=[
            pltpu.VMEM((N,), jnp.int32),
            pltpu.VMEM((CHUNK, D), jnp.float32),
            pltpu.VMEM((CHUNK,), jnp.int32),
            pltpu.VMEM((D,), jnp.float32),
            pltpu.VMEM((D,), jnp.float32),
            pltpu.VMEM((CPW, DP), jnp.float32),
        ],
    )(embs, ids, ln_gamma, ln_beta)


def _combine_body(part_ref, pers_ref, out_ref):
    p = part_ref[:NUM_CLASSES, :]
    counts = jnp.sum(p[:, D:], axis=-1, keepdims=True)
    out_ref[...] = p[:, :D] / jnp.maximum(counts, 1.0) + pers_ref[...]


def _combine(part, personal_table):
    return pl.pallas_call(
        _combine_body,
        out_shape=jax.ShapeDtypeStruct((NUM_CLASSES, D), jnp.float32),
    )(part, personal_table)


def kernel(embs, class_ids, personal_table, ln_gamma, ln_beta):
    ids = class_ids.astype(jnp.int32)
    part = _sc_segment_ln(embs, ids, ln_gamma, ln_beta)
    return _combine(part, personal_table)
